# baseline (speedup 1.0000x reference)
import functools

import jax
import jax.numpy as jnp
from jax import lax
from jax.experimental import pallas as pl
from jax.experimental.pallas import tpu as pltpu
from jax.experimental.pallas import tpu_sc as plsc

ROWS = 8192
D = 768
NC = 2
NS = 16
NW = NC * NS
ROWS_PER_W = ROWS // NW
CHUNK = 16
NCHUNK = ROWS_PER_W // CHUNK
NBUF = 10


@functools.partial(
    pl.kernel,
    mesh=plsc.VectorSubcoreMesh(core_axis_name="c", subcore_axis_name="s"),
    out_type=jax.ShapeDtypeStruct((ROWS, D), jnp.float32),
    scratch_types=(
        [pltpu.VMEM((CHUNK, D), jnp.float32) for _ in range(NBUF)]
        + [pltpu.SemaphoreType.DMA for _ in range(2 * NBUF)]
    ),
)
def _sc_copy(emb_hbm, out_hbm, *scratch):
    bufs = scratch[:NBUF]
    rsems = scratch[NBUF : 2 * NBUF]
    wsems = scratch[2 * NBUF :]
    wid = lax.axis_index("s") * NC + lax.axis_index("c")
    base = wid * ROWS_PER_W

    def rd(i, b):
        return pltpu.async_copy(
            emb_hbm.at[pl.ds(base + i * CHUNK, CHUNK)], bufs[b], rsems[b]
        )

    def wr(i, b):
        return pltpu.async_copy(
            bufs[b], out_hbm.at[pl.ds(base + i * CHUNK, CHUNK)], wsems[b]
        )

    reads = [None] * NBUF
    writes = [None] * NBUF
    for i in range(min(NBUF, NCHUNK)):
        reads[i] = rd(i, i)
    for i in range(NCHUNK):
        b = i % NBUF
        reads[b].wait()
        writes[b] = wr(i, b)
        nxt = i + NBUF
        if nxt < NCHUNK:
            writes[b].wait()
            reads[b] = rd(nxt, b)
    for i in range(max(0, NCHUNK - NBUF), NCHUNK):
        writes[i % NBUF].wait()


def kernel(x, emb):
    del x
    return _sc_copy(emb)


# speedup vs baseline: 1.0315x; 1.0315x over previous
import functools

import jax
import jax.numpy as jnp
from jax import lax
from jax.experimental import pallas as pl
from jax.experimental.pallas import tpu as pltpu
from jax.experimental.pallas import tpu_sc as plsc

ROWS = 8192
D = 768
NC = 2
NS = 16
NW = NC * NS
ROWS_PER_W = ROWS // NW
CHUNK = 32
NCHUNK = ROWS_PER_W // CHUNK
NBUF = 5


@functools.partial(
    pl.kernel,
    mesh=plsc.VectorSubcoreMesh(core_axis_name="c", subcore_axis_name="s"),
    out_type=jax.ShapeDtypeStruct((ROWS, D), jnp.float32),
    scratch_types=(
        [pltpu.VMEM((CHUNK, D), jnp.float32) for _ in range(NBUF)]
        + [pltpu.SemaphoreType.DMA for _ in range(2 * NBUF)]
    ),
)
def _sc_copy(emb_hbm, out_hbm, *scratch):
    bufs = scratch[:NBUF]
    rsems = scratch[NBUF : 2 * NBUF]
    wsems = scratch[2 * NBUF :]
    wid = lax.axis_index("s") * NC + lax.axis_index("c")
    base = wid * ROWS_PER_W

    def rd(i, b):
        return pltpu.async_copy(
            emb_hbm.at[pl.ds(base + i * CHUNK, CHUNK)], bufs[b], rsems[b]
        )

    def wr(i, b):
        return pltpu.async_copy(
            bufs[b], out_hbm.at[pl.ds(base + i * CHUNK, CHUNK)], wsems[b]
        )

    reads = [None] * NBUF
    writes = [None] * NBUF
    for i in range(min(NBUF, NCHUNK)):
        reads[i] = rd(i, i)
    for i in range(NCHUNK):
        b = i % NBUF
        reads[b].wait()
        writes[b] = wr(i, b)
        nxt = i + NBUF
        if nxt < NCHUNK:
            writes[b].wait()
            reads[b] = rd(nxt, b)
    for i in range(max(0, NCHUNK - NBUF), NCHUNK):
        writes[i % NBUF].wait()


def kernel(x, emb):
    del x
    return _sc_copy(emb)
